# baseline (speedup 1.0000x reference)
import functools

import jax
import jax.numpy as jnp
from jax import lax
from jax.experimental import pallas as pl
from jax.experimental.pallas import tpu as pltpu
from jax.experimental.pallas import tpu_sc as plsc

VOCAB = 1000000
EMBED = 32
INPUT_SIZE = 50
BATCH = 16384

NUM_CORES = 2
NUM_SUBCORES = 16
NW = NUM_CORES * NUM_SUBCORES
ROWS_PER_W = BATCH // NW
CHUNK = 64
NCHUNKS = ROWS_PER_W // CHUNK
IDX_PER_CHUNK = CHUNK * INPUT_SIZE
FLAT = INPUT_SIZE * EMBED
RB = 8
NB = CHUNK // RB
HIMASK = -65536


def _disc_body(x_hbm, tab_hbm, w_hbm, b_hbm, part_hbm,
               idx_a, idx_b, buf_a, buf_b, wv, bv, partials, sem_a, sem_b):
    cid = lax.axis_index("c")
    sid = lax.axis_index("s")
    wid = sid * NUM_CORES + cid

    pltpu.sync_copy(w_hbm, wv)
    pltpu.sync_copy(b_hbm, bv)

    def fire(idx_ref, buf_ref, sem, c):
        xoff = wid * (NCHUNKS * IDX_PER_CHUNK) + c * IDX_PER_CHUNK
        pltpu.sync_copy(x_hbm.at[pl.ds(xoff, IDX_PER_CHUNK)], idx_ref)
        pltpu.async_copy(tab_hbm.at[idx_ref], buf_ref, sem)

    def drain(buf_ref, sem):
        pltpu.make_async_copy(
            tab_hbm.at[pl.ds(0, IDX_PER_CHUNK)], buf_ref, sem
        ).wait()

    def compute(buf_ref, c):
        bias = bv[...]

        def blk_body(t, bcarry):
            r0 = t * RB

            def j_body(j, accs):
                w0 = wv[pl.ds(j * 32, 16)]
                w1 = wv[pl.ds(j * 32 + 16, 16)]
                out = []
                for rr in range(RB):
                    g = (r0 + rr) * INPUT_SIZE + j
                    bits = buf_ref[g, pl.ds(0, 16)]
                    ev = plsc.bitcast(bits << 16, jnp.float32)
                    od = plsc.bitcast(bits & HIMASK, jnp.float32)
                    out.append(accs[2 * rr] + ev * w0)
                    out.append(accs[2 * rr + 1] + od * w1)
                return tuple(out)

            zero = jnp.zeros((16,), jnp.float32)
            accs = lax.fori_loop(0, INPUT_SIZE, j_body, (zero,) * (2 * RB))
            for rr in range(RB):
                partials[pl.ds((r0 + rr) * 16, 16)] = (
                    accs[2 * rr] + accs[2 * rr + 1] + bias
                )
            return bcarry

        lax.fori_loop(0, NB, blk_body, 0)
        row0 = wid * ROWS_PER_W + c * CHUNK
        pltpu.sync_copy(partials, part_hbm.at[pl.ds(row0 * 16, CHUNK * 16)])

    fire(idx_a, buf_a, sem_a, 0)

    def m_body(m, carry):
        fire(idx_b, buf_b, sem_b, 2 * m + 1)
        drain(buf_a, sem_a)
        compute(buf_a, 2 * m)

        @pl.when(m < NCHUNKS // 2 - 1)
        def _():
            fire(idx_a, buf_a, sem_a, 2 * m + 2)

        drain(buf_b, sem_b)
        compute(buf_b, 2 * m + 1)
        return carry

    lax.fori_loop(0, NCHUNKS // 2, m_body, 0)


def _finalize_body(p_ref, o_ref):
    z = jnp.sum(p_ref[...], axis=1, keepdims=True)
    o_ref[...] = 1.0 / (1.0 + jnp.exp(-z))


def kernel(x, table, W, b):
    xf = x.astype(jnp.int32).reshape(BATCH * INPUT_SIZE)
    t16 = jax.lax.bitcast_convert_type(
        table.astype(jnp.bfloat16).reshape(VOCAB, EMBED // 2, 2), jnp.int32
    )
    wr = W.reshape(INPUT_SIZE, EMBED).astype(jnp.float32)
    wf = jnp.concatenate([wr[:, 0::2], wr[:, 1::2]], axis=1).reshape(FLAT)
    b16 = jnp.where(jnp.arange(16) == 0, b[0].astype(jnp.float32), 0.0)

    mesh = plsc.VectorSubcoreMesh(core_axis_name="c", subcore_axis_name="s")
    sc = pl.kernel(
        _disc_body,
        out_type=jax.ShapeDtypeStruct((BATCH * 16,), jnp.float32),
        mesh=mesh,
        compiler_params=pltpu.CompilerParams(
            use_tc_tiling_on_sc=False, needs_layout_passes=False
        ),
        scratch_types=[
            pltpu.VMEM((IDX_PER_CHUNK,), jnp.int32),
            pltpu.VMEM((IDX_PER_CHUNK,), jnp.int32),
            pltpu.VMEM((IDX_PER_CHUNK, EMBED // 2), jnp.int32),
            pltpu.VMEM((IDX_PER_CHUNK, EMBED // 2), jnp.int32),
            pltpu.VMEM((FLAT,), jnp.float32),
            pltpu.VMEM((16,), jnp.float32),
            pltpu.VMEM((CHUNK * 16,), jnp.float32),
            pltpu.SemaphoreType.DMA,
            pltpu.SemaphoreType.DMA,
        ],
    )
    partials = sc(xf, t16, wf, b16).reshape(BATCH, 16)

    blk = 2048
    out = pl.pallas_call(
        _finalize_body,
        out_shape=jax.ShapeDtypeStruct((BATCH, 1), jnp.float32),
        grid=(BATCH // blk,),
        in_specs=[pl.BlockSpec((blk, 16), lambda i: (i, 0))],
        out_specs=pl.BlockSpec((blk, 1), lambda i: (i, 0)),
    )(partials)
    return out


# speedup vs baseline: 1.9657x; 1.9657x over previous
import functools

import jax
import jax.numpy as jnp
from jax import lax
from jax.experimental import pallas as pl
from jax.experimental.pallas import tpu as pltpu
from jax.experimental.pallas import tpu_sc as plsc

VOCAB = 1000000
EMBED = 32
INPUT_SIZE = 50
BATCH = 16384

NUM_CORES = 2
NUM_SUBCORES = 16
NW = NUM_CORES * NUM_SUBCORES
ROWS_PER_W = BATCH // NW
CHUNK = 32
NCHUNKS = ROWS_PER_W // CHUNK
IDX_PER_CHUNK = CHUNK * INPUT_SIZE
FLAT = INPUT_SIZE * EMBED
RB = 8
NB = CHUNK // RB


def _disc_body(x_hbm, tab_hbm, w_hbm, b_hbm, part_hbm,
               idx_a, idx_b, buf_a, buf_b, wv, bv, partials, sem_a, sem_b):
    cid = lax.axis_index("c")
    sid = lax.axis_index("s")
    wid = sid * NUM_CORES + cid

    pltpu.sync_copy(w_hbm, wv)
    pltpu.sync_copy(b_hbm, bv)

    def fire(idx_ref, buf_ref, sem, c):
        xoff = wid * (NCHUNKS * IDX_PER_CHUNK) + c * IDX_PER_CHUNK
        pltpu.sync_copy(x_hbm.at[pl.ds(xoff, IDX_PER_CHUNK)], idx_ref)
        pltpu.async_copy(tab_hbm.at[idx_ref], buf_ref, sem)

    def drain(buf_ref, sem):
        pltpu.make_async_copy(
            tab_hbm.at[pl.ds(0, IDX_PER_CHUNK)], buf_ref, sem
        ).wait()

    def compute(buf_ref, c):
        bias = bv[...]

        def blk_body(t, bcarry):
            r0 = t * RB

            def j_body(j, accs):
                w0 = wv[pl.ds(j * 32, 16)]
                w1 = wv[pl.ds(j * 32 + 16, 16)]
                out = []
                for rr in range(RB):
                    g = (r0 + rr) * INPUT_SIZE + j
                    out.append(accs[2 * rr] + buf_ref[g, pl.ds(0, 16)] * w0)
                    out.append(accs[2 * rr + 1] + buf_ref[g, pl.ds(16, 16)] * w1)
                return tuple(out)

            zero = jnp.zeros((16,), jnp.float32)
            accs = lax.fori_loop(0, INPUT_SIZE, j_body, (zero,) * (2 * RB))
            for rr in range(RB):
                partials[pl.ds((r0 + rr) * 16, 16)] = (
                    accs[2 * rr] + accs[2 * rr + 1] + bias
                )
            return bcarry

        lax.fori_loop(0, NB, blk_body, 0)
        row0 = wid * ROWS_PER_W + c * CHUNK
        pltpu.sync_copy(partials, part_hbm.at[pl.ds(row0 * 16, CHUNK * 16)])

    fire(idx_a, buf_a, sem_a, 0)

    def m_body(m, carry):
        fire(idx_b, buf_b, sem_b, 2 * m + 1)
        drain(buf_a, sem_a)
        compute(buf_a, 2 * m)

        @pl.when(m < NCHUNKS // 2 - 1)
        def _():
            fire(idx_a, buf_a, sem_a, 2 * m + 2)

        drain(buf_b, sem_b)
        compute(buf_b, 2 * m + 1)
        return carry

    lax.fori_loop(0, NCHUNKS // 2, m_body, 0)


def _finalize_body(p_ref, o_ref):
    z = jnp.sum(p_ref[...], axis=1, keepdims=True)
    o_ref[...] = 1.0 / (1.0 + jnp.exp(-z))


def kernel(x, table, W, b):
    xf = x.astype(jnp.int32).reshape(BATCH * INPUT_SIZE)
    t128 = lax.optimization_barrier(table.reshape(VOCAB // 4, EMBED * 4))
    t2 = t128.reshape(VOCAB, EMBED)
    wf = W.reshape(FLAT).astype(jnp.float32)
    b16 = jnp.where(jnp.arange(16) == 0, b[0].astype(jnp.float32), 0.0)

    mesh = plsc.VectorSubcoreMesh(core_axis_name="c", subcore_axis_name="s")
    sc = pl.kernel(
        _disc_body,
        out_type=jax.ShapeDtypeStruct((BATCH * 16,), jnp.float32),
        mesh=mesh,
        compiler_params=pltpu.CompilerParams(
            use_tc_tiling_on_sc=False, needs_layout_passes=False
        ),
        scratch_types=[
            pltpu.VMEM((IDX_PER_CHUNK,), jnp.int32),
            pltpu.VMEM((IDX_PER_CHUNK,), jnp.int32),
            pltpu.VMEM((IDX_PER_CHUNK, EMBED), jnp.float32),
            pltpu.VMEM((IDX_PER_CHUNK, EMBED), jnp.float32),
            pltpu.VMEM((FLAT,), jnp.float32),
            pltpu.VMEM((16,), jnp.float32),
            pltpu.VMEM((CHUNK * 16,), jnp.float32),
            pltpu.SemaphoreType.DMA,
            pltpu.SemaphoreType.DMA,
        ],
    )
    partials = sc(xf, t2, wf, b16).reshape(BATCH, 16)

    blk = 2048
    out = pl.pallas_call(
        _finalize_body,
        out_shape=jax.ShapeDtypeStruct((BATCH, 1), jnp.float32),
        grid=(BATCH // blk,),
        in_specs=[pl.BlockSpec((blk, 16), lambda i: (i, 0))],
        out_specs=pl.BlockSpec((blk, 1), lambda i: (i, 0)),
    )(partials)
    return out
